# Initial kernel scaffold; baseline (speedup 1.0000x reference)
import jax
import jax.numpy as jnp
from jax.experimental import pallas as pl


def kernel(x, num_nodes, edge_index, edge_attr, mask, W, b):
    raise NotImplementedError("write your pallas kernel here")


# speedup vs baseline: 3.9879x; 3.9879x over previous
import functools

import jax
import jax.numpy as jnp
from jax import lax
from jax.experimental import pallas as pl
from jax.experimental.pallas import tpu as pltpu
from jax.experimental.pallas import tpu_sc as plsc

N = 100000
E = 3200000
D = 16

NC = 2
NS = 16
NW = NC * NS

N_PAD = 100352
ROWS_PER_TILE = N_PAD // NS

S = 80
K = 8
C = S * K
N_CHUNKS = E // C

_mesh = plsc.VectorSubcoreMesh(core_axis_name="c", subcore_axis_name="s")
_sc_params = pltpu.CompilerParams(use_tc_tiling_on_sc=False)


def _worker_id():
    return lax.axis_index("s") * NC + lax.axis_index("c")


@functools.partial(
    pl.kernel,
    out_type=[
        jax.ShapeDtypeStruct((NC, N_PAD, D), jnp.float32),
        jax.ShapeDtypeStruct((NC, N_PAD), jnp.float32),
    ],
    mesh=_mesh,
    scratch_types=[
        pltpu.VMEM((K, S), jnp.int32),
        pltpu.VMEM((C,), jnp.float32),
        pltpu.VMEM((C, D), jnp.float32),
        pltpu.VMEM_SHARED((N_PAD, D), jnp.float32),
        pltpu.VMEM_SHARED((N_PAD,), jnp.float32),
    ],
    compiler_params=_sc_params,
)
def _sc_scatter(col2d, mask_h, wgt_h, out_sum, out_deg,
                col_v, mask_v, val_v, acc_sum, acc_deg):
    cid = lax.axis_index("c")
    sid = lax.axis_index("s")
    wid = _worker_id()

    zero16 = jnp.zeros((D,), jnp.float32)

    def zrow(i, _):
        val_v[i, :] = zero16
        mask_v[pl.ds(i * 16, 16)] = jnp.zeros((16,), jnp.float32)
        return 0
    lax.fori_loop(0, C // 16, zrow, 0)

    def zrow2(i, _):
        val_v[i, :] = zero16
        return 0
    lax.fori_loop(C // 16, C, zrow2, 0)
    r0 = sid * ROWS_PER_TILE
    for j in range(ROWS_PER_TILE // C):
        pltpu.sync_copy(val_v.at[pl.ds(0, C), :],
                        acc_sum.at[pl.ds(r0 + j * C, C), :])
        pltpu.sync_copy(mask_v.at[pl.ds(0, C)],
                        acc_deg.at[pl.ds(r0 + j * C, C)])
    _rem = ROWS_PER_TILE - (ROWS_PER_TILE // C) * C
    _rbase = r0 + (ROWS_PER_TILE // C) * C
    pltpu.sync_copy(val_v.at[pl.ds(0, _rem), :],
                    acc_sum.at[pl.ds(_rbase, _rem), :])
    pltpu.sync_copy(mask_v.at[pl.ds(0, _rem)],
                    acc_deg.at[pl.ds(_rbase, _rem)])
    plsc.subcore_barrier()

    niter = jnp.where(wid < N_CHUNKS % NW, N_CHUNKS // NW + 1, N_CHUNKS // NW)

    def chunk_body(i, _):
        chunk = wid + i * NW
        ebase = chunk * C
        crow = chunk * K
        pltpu.sync_copy(col2d.at[pl.ds(crow, K), :], col_v)
        pltpu.sync_copy(mask_h.at[pl.ds(ebase, C)], mask_v)
        pltpu.sync_copy(wgt_h.at[pl.ds(ebase, C), :], val_v)

        for j in range(K):
            idx = col_v.at[j]
            pltpu.sync_copy(val_v.at[pl.ds(j * S, S), :],
                            acc_sum.at[idx], add=True)
            pltpu.sync_copy(mask_v.at[pl.ds(j * S, S)],
                            acc_deg.at[idx], add=True)
        return 0

    lax.fori_loop(0, niter, chunk_body, 0)
    plsc.subcore_barrier()

    pltpu.sync_copy(acc_sum.at[pl.ds(r0, ROWS_PER_TILE), :],
                    out_sum.at[cid, pl.ds(r0, ROWS_PER_TILE), :])
    pltpu.sync_copy(acc_deg.at[pl.ds(r0, ROWS_PER_TILE)],
                    out_deg.at[cid, pl.ds(r0, ROWS_PER_TILE)])


@functools.partial(
    pl.kernel,
    out_type=jax.ShapeDtypeStruct((E, D), jnp.float32),
    mesh=_mesh,
    scratch_types=[
        pltpu.VMEM((K, S), jnp.int32),
        pltpu.VMEM((K, S), jnp.int32),
        pltpu.VMEM((C, D), jnp.float32),
    ],
    compiler_params=_sc_params,
)
def _sc_gather(row2d, col2d, p1_h, p2_h, q_h, out_h, row_v, col_v, gbuf):
    wid = _worker_id()
    niter = jnp.where(wid < N_CHUNKS % NW, N_CHUNKS // NW + 1, N_CHUNKS // NW)

    def chunk_body(i, _):
        chunk = wid + i * NW
        ebase = chunk * C
        crow = chunk * K
        pltpu.sync_copy(row2d.at[pl.ds(crow, K), :], row_v)
        pltpu.sync_copy(col2d.at[pl.ds(crow, K), :], col_v)
        pltpu.sync_copy(q_h.at[pl.ds(ebase, C), :], gbuf)
        for j in range(K):
            dst = gbuf.at[pl.ds(j * S, S), :]
            pltpu.sync_copy(p1_h.at[row_v.at[j]], dst, add=True)
            pltpu.sync_copy(p2_h.at[col_v.at[j]], dst, add=True)
        pltpu.sync_copy(gbuf, out_h.at[pl.ds(ebase, C), :])
        return 0

    lax.fori_loop(0, niter, chunk_body, 0)


_BN = 5000


def _tc_node_body(p0, p1, d0, d1, x, w1t, w2t, nrep, o1, o2):
    s = p0[...] + p1[...]
    deg = d0[...] + d1[...]
    nr1 = s / (deg + 1.0)
    xb = x[...]
    nrep[:, :D] = nr1
    nrep[:, D:] = xb
    rep = jnp.concatenate([nr1, xb], axis=1)
    o1[...] = jnp.dot(rep, w1t[...], preferred_element_type=jnp.float32)
    o2[...] = jnp.dot(rep, w2t[...], preferred_element_type=jnp.float32)


def _tc_node(p0, p1, d0, d1, x, w1t, w2t):
    return pl.pallas_call(
        _tc_node_body,
        grid=(N // _BN,),
        in_specs=[
            pl.BlockSpec((_BN, D), lambda i: (i, 0)),
            pl.BlockSpec((_BN, D), lambda i: (i, 0)),
            pl.BlockSpec((_BN, 1), lambda i: (i, 0)),
            pl.BlockSpec((_BN, 1), lambda i: (i, 0)),
            pl.BlockSpec((_BN, D), lambda i: (i, 0)),
            pl.BlockSpec((2 * D, D), lambda i: (0, 0)),
            pl.BlockSpec((2 * D, D), lambda i: (0, 0)),
        ],
        out_specs=[
            pl.BlockSpec((_BN, 2 * D), lambda i: (i, 0)),
            pl.BlockSpec((_BN, D), lambda i: (i, 0)),
            pl.BlockSpec((_BN, D), lambda i: (i, 0)),
        ],
        out_shape=[
            jax.ShapeDtypeStruct((N, 2 * D), jnp.float32),
            jax.ShapeDtypeStruct((N, D), jnp.float32),
            jax.ShapeDtypeStruct((N, D), jnp.float32),
        ],
    )(p0, p1, d0, d1, x, w1t, w2t)


_BE = 12800


def _tc_edge_body(attr, mask, w3t, b, wgt, q):
    a = attr[...]
    wgt[...] = a * mask[...]
    q[...] = jnp.dot(a, w3t[...], preferred_element_type=jnp.float32) + b[...]


def _tc_edge(attr, mask2d, w3t, b2):
    return pl.pallas_call(
        _tc_edge_body,
        grid=(E // _BE,),
        in_specs=[
            pl.BlockSpec((_BE, D), lambda i: (i, 0)),
            pl.BlockSpec((_BE, 1), lambda i: (i, 0)),
            pl.BlockSpec((D, D), lambda i: (0, 0)),
            pl.BlockSpec((1, D), lambda i: (0, 0)),
        ],
        out_specs=[
            pl.BlockSpec((_BE, D), lambda i: (i, 0)),
            pl.BlockSpec((_BE, D), lambda i: (i, 0)),
        ],
        out_shape=[
            jax.ShapeDtypeStruct((E, D), jnp.float32),
            jax.ShapeDtypeStruct((E, D), jnp.float32),
        ],
    )(attr, mask2d, w3t, b2)


def kernel(x, num_nodes, edge_index, edge_attr, mask, W, b):
    row2d = edge_index[0].reshape(E // S, S)
    col2d = edge_index[1].reshape(E // S, S)

    w1t = W[:, : 2 * D].T
    w2t = W[:, 2 * D : 4 * D].T
    w3t = W[:, 4 * D :].T

    weighted, q = _tc_edge(edge_attr, mask[:, None], w3t, b[None, :])
    part_sum, part_deg = _sc_scatter(col2d, mask, weighted)
    node_rep, p1, p2 = _tc_node(
        part_sum[0], part_sum[1],
        part_deg[0][:, None], part_deg[1][:, None],
        x, w1t, w2t)
    edge_rep = _sc_gather(row2d, col2d, p1, p2, q)
    return (node_rep, edge_rep)
